# Initial kernel scaffold; baseline (speedup 1.0000x reference)
import jax
import jax.numpy as jnp
from jax.experimental import pallas as pl


def kernel(x, weight):
    raise NotImplementedError("write your pallas kernel here")


# speedup vs baseline: 8.1941x; 8.1941x over previous
import functools

import jax
import jax.numpy as jnp
from jax import lax
from jax.experimental import pallas as pl
from jax.experimental.pallas import tpu as pltpu
from jax.experimental.pallas import tpu_sc as plsc

NUM_EMBEDDINGS = 100000
EMBEDDING_DIM = 128

_INFO = plsc.get_sparse_core_info()
_NC = _INFO.num_cores
_NS = _INFO.num_subcores
_NW = _NC * _NS

_IDX_COLS = 128
_K = 4


def _gather_body(weight_hbm, idx_hbm, out_hbm, idx_v, rows_v, sem,
                 *, rows_per_worker):
    wid = lax.axis_index("s") * _NC + lax.axis_index("c")
    row_base = wid * rows_per_worker
    n_chunks = rows_per_worker // _K

    def chunk(i):
        r0 = row_base + i * _K
        pltpu.sync_copy(idx_hbm.at[pl.ds(r0, _K)], idx_v)
        copies = []
        for j in range(_K):
            copies.append(
                pltpu.async_copy(
                    weight_hbm.at[idx_v.at[j]],
                    rows_v.at[pl.ds(j * _IDX_COLS, _IDX_COLS)],
                    sem,
                )
            )
        for c in copies:
            c.wait()
        pltpu.sync_copy(rows_v, out_hbm.at[pl.ds(r0 * _IDX_COLS, _K * _IDX_COLS)])

    pl.loop(0, n_chunks)(chunk)


def kernel(x, weight):
    b, t = x.shape
    n_idx = b * t
    assert n_idx % (_NW * _IDX_COLS * _K) == 0
    idx_rows = n_idx // _IDX_COLS
    rows_per_worker = idx_rows // _NW

    xf = x.reshape(idx_rows, _IDX_COLS).astype(jnp.int32)

    mesh = plsc.VectorSubcoreMesh(core_axis_name="c", subcore_axis_name="s")
    body = functools.partial(_gather_body, rows_per_worker=rows_per_worker)
    out = pl.kernel(
        body,
        mesh=mesh,
        out_type=jax.ShapeDtypeStruct((n_idx, EMBEDDING_DIM), jnp.float32),
        scratch_types=[
            pltpu.VMEM((_K, _IDX_COLS), jnp.int32),
            pltpu.VMEM((_K * _IDX_COLS, EMBEDDING_DIM), jnp.float32),
            pltpu.SemaphoreType.DMA,
        ],
    )(weight, xf)
    return out.reshape(b, t, EMBEDDING_DIM)


# speedup vs baseline: 9.2458x; 1.1283x over previous
import functools

import jax
import jax.numpy as jnp
from jax import lax
from jax.experimental import pallas as pl
from jax.experimental.pallas import tpu as pltpu
from jax.experimental.pallas import tpu_sc as plsc

NUM_EMBEDDINGS = 100000
EMBEDDING_DIM = 128

_INFO = plsc.get_sparse_core_info()
_NC = _INFO.num_cores
_NS = _INFO.num_subcores
_NW = _NC * _NS

_IDX_COLS = 128
_D = 5


def _gather_body(weight_hbm, idx_hbm, out_hbm, idx_v, rows_v,
                 gsems, osems, *, rows_per_worker):
    wid = lax.axis_index("s") * _NC + lax.axis_index("c")
    row_base = wid * rows_per_worker
    n_outer = rows_per_worker // _D

    pltpu.sync_copy(idx_hbm.at[pl.ds(row_base, rows_per_worker)], idx_v)

    def fire_gather(c, b):
        return pltpu.async_copy(
            weight_hbm.at[idx_v.at[c]], rows_v.at[b], gsems[b])

    def fire_out(c, b):
        return pltpu.async_copy(
            rows_v.at[b],
            out_hbm.at[pl.ds((row_base + c) * _IDX_COLS, _IDX_COLS)],
            osems[b])

    for b in range(_D - 1):
        fire_gather(b, b)

    def outer(i):
        for b in range(_D):
            c = i * _D + b
            nb = (b + _D - 1) % _D

            @pl.when(c + _D - 1 < rows_per_worker)
            def _():
                @pl.when(c >= 1)
                def _():
                    pltpu.make_async_copy(
                        rows_v.at[nb],
                        out_hbm.at[pl.ds(row_base * _IDX_COLS, _IDX_COLS)],
                        osems[nb],
                    ).wait()
                fire_gather(c + _D - 1, nb)

            pltpu.make_async_copy(
                weight_hbm.at[idx_v.at[c]], rows_v.at[b], gsems[b]).wait()
            fire_out(c, b)

    pl.loop(0, n_outer)(outer)

    for b in range(_D):
        pltpu.make_async_copy(
            rows_v.at[b],
            out_hbm.at[pl.ds(row_base * _IDX_COLS, _IDX_COLS)],
            osems[b],
        ).wait()


def kernel(x, weight):
    b, t = x.shape
    n_idx = b * t
    assert n_idx % (_NW * _IDX_COLS * _D) == 0
    idx_rows = n_idx // _IDX_COLS
    rows_per_worker = idx_rows // _NW

    xf = x.reshape(idx_rows, _IDX_COLS).astype(jnp.int32)

    mesh = plsc.VectorSubcoreMesh(core_axis_name="c", subcore_axis_name="s")
    body = functools.partial(_gather_body, rows_per_worker=rows_per_worker)
    out = pl.kernel(
        body,
        mesh=mesh,
        out_type=jax.ShapeDtypeStruct((n_idx, EMBEDDING_DIM), jnp.float32),
        scratch_types=[
            pltpu.VMEM((rows_per_worker, _IDX_COLS), jnp.int32),
            pltpu.VMEM((_D, _IDX_COLS, EMBEDDING_DIM), jnp.float32),
            [pltpu.SemaphoreType.DMA] * _D,
            [pltpu.SemaphoreType.DMA] * _D,
        ],
    )(weight, xf)
    return out.reshape(b, t, EMBEDDING_DIM)


# speedup vs baseline: 9.2500x; 1.0005x over previous
import functools

import jax
import jax.numpy as jnp
from jax import lax
from jax.experimental import pallas as pl
from jax.experimental.pallas import tpu as pltpu
from jax.experimental.pallas import tpu_sc as plsc

NUM_EMBEDDINGS = 100000
EMBEDDING_DIM = 128

_INFO = plsc.get_sparse_core_info()
_NC = _INFO.num_cores
_NS = _INFO.num_subcores
_NW = _NC * _NS

_IDX_COLS = 128
_D = 5
_G = 2


def _gather_body(weight_hbm, idx_hbm, out_hbm, idx_v, rows_v,
                 gsems, osems, *, rows_per_worker):
    wid = lax.axis_index("s") * _NC + lax.axis_index("c")
    row_base = wid * rows_per_worker
    n_outer = rows_per_worker // _D

    pltpu.sync_copy(idx_hbm.at[pl.ds(row_base, rows_per_worker)], idx_v)

    def fire_gather(c, b):
        return pltpu.async_copy(
            weight_hbm.at[idx_v.at[c]], rows_v.at[b], gsems[b])

    def fire_out(c, b):
        return pltpu.async_copy(
            rows_v.at[b],
            out_hbm.at[pl.ds((row_base + c) * _IDX_COLS, _IDX_COLS)],
            osems[b])

    for b in range(_G):
        fire_gather(b, b)

    def outer(i):
        for b in range(_D):
            c = i * _D + b
            nb = (b + _G) % _D

            @pl.when(c + _G < rows_per_worker)
            def _():
                @pl.when(c + _G >= _D)
                def _():
                    pltpu.make_async_copy(
                        rows_v.at[nb],
                        out_hbm.at[pl.ds(row_base * _IDX_COLS, _IDX_COLS)],
                        osems[nb],
                    ).wait()
                fire_gather(c + _G, nb)

            pltpu.make_async_copy(
                weight_hbm.at[idx_v.at[c]], rows_v.at[b], gsems[b]).wait()
            fire_out(c, b)

    pl.loop(0, n_outer)(outer)

    for b in range(_D):
        pltpu.make_async_copy(
            rows_v.at[b],
            out_hbm.at[pl.ds(row_base * _IDX_COLS, _IDX_COLS)],
            osems[b],
        ).wait()


def kernel(x, weight):
    b, t = x.shape
    n_idx = b * t
    assert n_idx % (_NW * _IDX_COLS * _D) == 0
    idx_rows = n_idx // _IDX_COLS
    rows_per_worker = idx_rows // _NW

    xf = x.reshape(idx_rows, _IDX_COLS).astype(jnp.int32)

    mesh = plsc.VectorSubcoreMesh(core_axis_name="c", subcore_axis_name="s")
    body = functools.partial(_gather_body, rows_per_worker=rows_per_worker)
    out = pl.kernel(
        body,
        mesh=mesh,
        out_type=jax.ShapeDtypeStruct((n_idx, EMBEDDING_DIM), jnp.float32),
        scratch_types=[
            pltpu.VMEM((rows_per_worker, _IDX_COLS), jnp.int32),
            pltpu.VMEM((_D, _IDX_COLS, EMBEDDING_DIM), jnp.float32),
            [pltpu.SemaphoreType.DMA] * _D,
            [pltpu.SemaphoreType.DMA] * _D,
        ],
    )(weight, xf)
    return out.reshape(b, t, EMBEDDING_DIM)
